# baseline (speedup 1.0000x reference)
import functools

import jax
import jax.numpy as jnp
from jax import lax
from jax.experimental import pallas as pl
from jax.experimental.pallas import tpu as pltpu
from jax.experimental.pallas import tpu_sc as plsc

_CAT = (1000, 100000, 1000, 48, 2)
_EMB = (4, 21, 1, 1, 2)
_OFF = (0, 4, 25, 26, 27)
_SMALL = ((0, 0), (2, 25), (3, 26), (4, 27))
_DOUT = 29
_B = 16384
_NC, _NS, _L = 2, 16, 16
_NW = _NC * _NS
_BPW = _B // _NW
_W1P = 24

_mesh = plsc.VectorSubcoreMesh(
    core_axis_name="c", subcore_axis_name="s", num_cores=_NC, num_subcores=_NS
)


@functools.partial(
    pl.kernel,
    out_type=jax.ShapeDtypeStruct((_B * _DOUT,), jnp.float32),
    mesh=_mesh,
    scratch_types=[
        pltpu.VMEM((4, _BPW), jnp.int32),
        pltpu.VMEM((_BPW,), jnp.int32),
        pltpu.VMEM((_BPW, _W1P), jnp.float32),
        pltpu.VMEM((_BPW * _DOUT,), jnp.float32),
        pltpu.VMEM((_CAT[0], _EMB[0]), jnp.float32),
        pltpu.VMEM((_CAT[2], _EMB[2]), jnp.float32),
        pltpu.VMEM((_CAT[3], _EMB[3]), jnp.float32),
        pltpu.VMEM((_CAT[4], _EMB[4]), jnp.float32),
        pltpu.SemaphoreType.DMA,
    ],
    compiler_params=pltpu.CompilerParams(
        needs_layout_passes=False, use_tc_tiling_on_sc=False),
)
def _emb_kernel(xt_hbm, t0_hbm, t1_hbm, t2_hbm, t3_hbm, t4_hbm, out_hbm,
                xi_v, idx1_v, rows1_v, out_v, t0_v, t2_v, t3_v, t4_v, sem):
    wid = lax.axis_index("s") * _NC + lax.axis_index("c")
    base = wid * _BPW

    pltpu.sync_copy(xt_hbm.at[1, pl.ds(base, _BPW)], idx1_v)
    for r, (t, _) in enumerate(_SMALL):
        pltpu.sync_copy(xt_hbm.at[t, pl.ds(base, _BPW)], xi_v.at[r])

    copies = [pltpu.async_copy(t1_hbm.at[idx1_v], rows1_v, sem)]

    pltpu.sync_copy(t0_hbm, t0_v)
    pltpu.sync_copy(t2_hbm, t2_v)
    pltpu.sync_copy(t3_hbm, t3_v)
    pltpu.sync_copy(t4_hbm, t4_v)

    iota = lax.iota(jnp.int32, _L)
    tv = {0: t0_v, 2: t2_v, 3: t3_v, 4: t4_v}

    def small_body(i, carry):
        rows = iota + i * _L
        flat = rows * _DOUT
        for r, (t, off) in enumerate(_SMALL):
            cidx = jnp.clip(xi_v[r, pl.ds(i * _L, _L)], 0, _CAT[t] - 1)
            for jj in range(_EMB[t]):
                v = plsc.load_gather(tv[t], [cidx, jnp.full((_L,), jj, jnp.int32)])
                plsc.store_scatter(out_v, [flat + (off + jj)], v)
        return carry

    lax.fori_loop(0, _BPW // _L, small_body, 0)

    for cp in copies:
        cp.wait()

    def shuffle_body(i, carry):
        rows = iota + i * _L
        flat = rows * _DOUT
        for c in range(_EMB[1]):
            v = plsc.load_gather(rows1_v, [rows, jnp.full((_L,), c, jnp.int32)])
            plsc.store_scatter(out_v, [flat + (_OFF[1] + c)], v)
        return carry

    lax.fori_loop(0, _BPW // _L, shuffle_body, 0)

    pltpu.sync_copy(out_v, out_hbm.at[pl.ds(base * _DOUT, _BPW * _DOUT)])


def kernel(x, table_0, table_1, table_2, table_3, table_4):
    xt = jnp.transpose(x.astype(jnp.int32))
    t1p = jnp.pad(table_1, ((0, 0), (0, _W1P - _EMB[1])))
    out = _emb_kernel(xt, table_0, t1p, table_2, table_3, table_4)
    return out.reshape(_B, _DOUT)


# speedup vs baseline: 1.3295x; 1.3295x over previous
import functools

import jax
import jax.numpy as jnp
from jax import lax
from jax.experimental import pallas as pl
from jax.experimental.pallas import tpu as pltpu
from jax.experimental.pallas import tpu_sc as plsc

_CAT = (1000, 100000, 1000, 48, 2)
_EMB = (4, 21, 1, 1, 2)
_OFF = (0, 4, 25, 26, 27)
_SMALL = ((0, 0), (2, 25), (3, 26), (4, 27))
_DOUT = 29
_B = 16384
_NC, _NS, _L = 2, 16, 16
_NW = _NC * _NS
_BPW = _B // _NW
_W1P = 128

_mesh = plsc.VectorSubcoreMesh(
    core_axis_name="c", subcore_axis_name="s", num_cores=_NC, num_subcores=_NS
)


@functools.partial(
    pl.kernel,
    out_type=jax.ShapeDtypeStruct((_B, _DOUT), jnp.float32),
    mesh=_mesh,
    scratch_types=[
        pltpu.VMEM((4, _BPW), jnp.int32),
        pltpu.VMEM((_BPW,), jnp.int32),
        pltpu.VMEM((_BPW, _W1P), jnp.float32),
        pltpu.VMEM((_BPW, _DOUT), jnp.float32),
        pltpu.VMEM((_CAT[0], _EMB[0]), jnp.float32),
        pltpu.VMEM((_CAT[2], _EMB[2]), jnp.float32),
        pltpu.VMEM((_CAT[3], _EMB[3]), jnp.float32),
        pltpu.VMEM((_CAT[4], _EMB[4]), jnp.float32),
        pltpu.SemaphoreType.DMA,
    ],
    compiler_params=pltpu.CompilerParams(
        needs_layout_passes=False, use_tc_tiling_on_sc=False),
)
def _emb_kernel(xt_hbm, t0_hbm, t1_hbm, t2_hbm, t3_hbm, t4_hbm, out_hbm,
                xi_v, idx1_v, rows1_v, out_v, t0_v, t2_v, t3_v, t4_v, sem):
    wid = lax.axis_index("s") * _NC + lax.axis_index("c")
    base = wid * _BPW

    pltpu.sync_copy(xt_hbm.at[1, pl.ds(base, _BPW)], idx1_v)
    for r, (t, _) in enumerate(_SMALL):
        pltpu.sync_copy(xt_hbm.at[t, pl.ds(base, _BPW)], xi_v.at[r])

    copies = [pltpu.async_copy(t1_hbm.at[idx1_v], rows1_v, sem)]

    pltpu.sync_copy(t0_hbm, t0_v)
    pltpu.sync_copy(t2_hbm, t2_v)
    pltpu.sync_copy(t3_hbm, t3_v)
    pltpu.sync_copy(t4_hbm, t4_v)

    iota = lax.iota(jnp.int32, _L)
    tv = {0: t0_v, 2: t2_v, 3: t3_v, 4: t4_v}

    def small_body(i, carry):
        rows = iota + i * _L
        for r, (t, off) in enumerate(_SMALL):
            cidx = jnp.clip(xi_v[r, pl.ds(i * _L, _L)], 0, _CAT[t] - 1)
            for jj in range(_EMB[t]):
                v = plsc.load_gather(tv[t], [cidx, jnp.full((_L,), jj, jnp.int32)])
                plsc.store_scatter(
                    out_v, [rows, jnp.full((_L,), off + jj, jnp.int32)], v)
        return carry

    lax.fori_loop(0, _BPW // _L, small_body, 0)

    for cp in copies:
        cp.wait()

    def shuffle_body(i, carry):
        rows = iota + i * _L
        for c in range(_EMB[1]):
            v = plsc.load_gather(rows1_v, [rows, jnp.full((_L,), c, jnp.int32)])
            plsc.store_scatter(
                out_v, [rows, jnp.full((_L,), _OFF[1] + c, jnp.int32)], v)
        return carry

    lax.fori_loop(0, _BPW // _L, shuffle_body, 0)

    pltpu.sync_copy(out_v, out_hbm.at[pl.ds(base, _BPW), :])


def kernel(x, table_0, table_1, table_2, table_3, table_4):
    xt = jnp.transpose(x.astype(jnp.int32))
    t1p = jnp.pad(table_1, ((0, 0), (0, _W1P - _EMB[1])))
    return _emb_kernel(xt, table_0, t1p, table_2, table_3, table_4)
